# baseline (speedup 1.0000x reference)
import functools

import jax
import jax.numpy as jnp
from jax import lax
from jax.experimental import pallas as pl
from jax.experimental.pallas import tpu as pltpu
from jax.experimental.pallas import tpu_sc as plsc

E, K, S, R, NH, TOPK = 8, 4, 64, 64, 8, 2
HW = 14
L = HW * HW
LP = 208
D = 256
HD = D // NH
EPAD = 128
NEG = -1e30


def _silu(v):
    return v / (1.0 + jnp.exp(-v))


def _softplus(v):
    return jnp.maximum(v, 0.0) + jnp.log(1.0 + jnp.exp(-jnp.abs(v)))


def _gate_body(x_ref, siw_ref, sib_ref, sow_ref, sob_ref,
               cqw_ref, cqb_ref, ckw_ref, ckb_ref, eq_ref, aw_ref):
    xv = x_ref[0]
    qkv = jax.lax.dot_general(xv, siw_ref[...], (((1,), (1,)), ((), ())),
                              preferred_element_type=jnp.float32) + sib_ref[...]
    scale = 1.0 / jnp.sqrt(jnp.float32(HD))
    outs = []
    for h in range(NH):
        q = qkv[:, h * HD:(h + 1) * HD]
        k = qkv[:, D + h * HD:D + (h + 1) * HD]
        v = qkv[:, 2 * D + h * HD:2 * D + (h + 1) * HD]
        s = jax.lax.dot_general(q, k, (((1,), (1,)), ((), ())),
                                preferred_element_type=jnp.float32) * scale
        s = s - jnp.max(s, axis=-1, keepdims=True)
        p = jnp.exp(s)
        p = p / jnp.sum(p, axis=-1, keepdims=True)
        outs.append(jnp.dot(p, v, preferred_element_type=jnp.float32))
    ctx0 = jnp.concatenate(outs, axis=1)
    ctx = jax.lax.dot_general(ctx0, sow_ref[...], (((1,), (1,)), ((), ())),
                              preferred_element_type=jnp.float32) + sob_ref[...]
    q2 = jax.lax.dot_general(ctx, cqw_ref[...], (((1,), (1,)), ((), ())),
                             preferred_element_type=jnp.float32) + cqb_ref[...]
    k2 = jax.lax.dot_general(eq_ref[...], ckw_ref[...], (((1,), (1,)), ((), ())),
                             preferred_element_type=jnp.float32) + ckb_ref[...]
    col = jax.lax.broadcasted_iota(jnp.int32, (L, EPAD), 1)
    acc = jnp.zeros((L, EPAD), jnp.float32)
    for h in range(NH):
        q2h = q2[:, h * HD:(h + 1) * HD]
        k2h = k2[:, h * HD:(h + 1) * HD]
        l2 = jax.lax.dot_general(q2h, k2h, (((1,), (1,)), ((), ())),
                                 preferred_element_type=jnp.float32) * scale
        l2 = jnp.where(col < E, l2, NEG)
        l2 = l2 - jnp.max(l2, axis=-1, keepdims=True)
        p2 = jnp.exp(l2)
        p2 = p2 / jnp.sum(p2, axis=-1, keepdims=True)
        acc = acc + p2
    aw_ref[...] = (jnp.sum(acc, axis=0) * (1.0 / (NH * L))).reshape(1, 1, EPAD)


def _route_sc_body(cap, nb, awT_hbm, out_hbm, aw_v, out_v):
    c0 = lax.axis_index("c") == 0
    s0 = lax.axis_index("s") == 0

    def vsum(v):
        idx = lax.iota(jnp.int32, nb)
        for k in (8, 4, 2, 1):
            v = v + v.at[jnp.bitwise_xor(idx, k)].get(
                mode="promise_in_bounds", unique_indices=True)
        return v

    @pl.when(jnp.logical_and(c0, s0))
    def _():
        pltpu.sync_copy(awT_hbm, aw_v)
        cols = [aw_v[e] for e in range(E)]
        m = cols[0]
        for e in range(1, E):
            m = jnp.maximum(m, cols[e])
        exs = [jnp.exp(c - m) for c in cols]
        ssum = exs[0]
        for e in range(1, E):
            ssum = ssum + exs[e]
        raw = [ex / ssum for ex in exs]

        def top2(vals):
            b1 = vals[0]
            i1 = jnp.zeros((nb,), jnp.float32)
            for e in range(1, E):
                upd = vals[e] > b1
                b1 = jnp.where(upd, vals[e], b1)
                i1 = jnp.where(upd, float(e), i1)
            b2 = jnp.full((nb,), -1.0, jnp.float32)
            i2 = jnp.zeros((nb,), jnp.float32)
            for e in range(E):
                upd = jnp.logical_and(i1 != float(e), vals[e] > b2)
                b2 = jnp.where(upd, vals[e], b2)
                i2 = jnp.where(upd, float(e), i2)
            return b1, i1, b2, i2

        _, i1, _, i2 = top2(raw)
        masks = [jnp.where(jnp.logical_or(i1 == float(e), i2 == float(e)),
                           1.0, 0.0) for e in range(E)]
        gates = []
        auxs = jnp.zeros((nb,), jnp.float32)
        for e in range(E):
            me = raw[e] * masks[e]
            gates.append(me / (vsum(me) + 1e-6) * cap)
            auxs = auxs + vsum(raw[e]) * vsum(masks[e])
        g1, j1, g2, j2 = top2(gates)
        e2v = jnp.exp(g2 - g1)
        t1 = 1.0 / (1.0 + e2v)
        t2 = e2v / (1.0 + e2v)
        out_v[0] = t1
        out_v[1] = t2
        out_v[2] = j1
        out_v[3] = j2
        out_v[4] = auxs * (0.01 * E / (nb * nb))
        zero = jnp.zeros((nb,), jnp.float32)
        out_v[5] = zero
        out_v[6] = zero
        out_v[7] = zero
        pltpu.sync_copy(out_v, out_hbm)


def _route_body(cap, nb, aw_ref, out_ref):
    aw = aw_ref[...]
    col = jax.lax.broadcasted_iota(jnp.int32, (nb, EPAD), 1)
    valid = col < E
    logits = jnp.where(valid, aw, NEG)
    m = jnp.max(logits, -1, keepdims=True)
    ex = jnp.exp(logits - m)
    raw = ex / jnp.sum(ex, -1, keepdims=True)

    def top1(vals):
        mv = jnp.max(vals, -1, keepdims=True)
        idx = jnp.min(jnp.where(vals == mv, col, 1 << 20), -1, keepdims=True)
        return mv, idx

    sel = jnp.where(valid, raw, -1.0)
    _, i1 = top1(sel)
    sel2 = jnp.where(col == i1, -1.0, sel)
    _, i2 = top1(sel2)
    maskf = ((col == i1) | (col == i2)).astype(jnp.float32)
    masked = raw * maskf
    colsum = jnp.sum(masked, axis=0, keepdims=True)
    gate = masked / (colsum + 1e-6) * cap
    gsel = jnp.where(valid, gate, -1.0)
    g1, j1 = top1(gsel)
    gsel2 = jnp.where(col == j1, -1.0, gsel)
    g2, j2 = top1(gsel2)
    e2 = jnp.exp(g2 - g1)
    t1 = 1.0 / (1.0 + e2)
    t2 = e2 / (1.0 + e2)
    rm = jnp.sum(raw, axis=0, keepdims=True) * (1.0 / nb)
    mm = jnp.sum(maskf, axis=0, keepdims=True) * (1.0 / nb)
    aux = 0.01 * jnp.sum(rm * mm) * E
    out = (jnp.where(col == 0, t1, 0.0)
           + jnp.where(col == 1, t2, 0.0)
           + jnp.where(col == 2, j1.astype(jnp.float32), 0.0)
           + jnp.where(col == 3, j2.astype(jnp.float32), 0.0)
           + jnp.where(col == 4, aux, 0.0))
    out_ref[...] = out


def _expert_body(pe_ref, x_ref, inw_ref, inb_ref, cw_ref, cb_ref, xpw_ref,
                 dtw_ref, dtb_ref, alog_ref, dp_ref, onw_ref, onb_ref,
                 lnw_ref, lnb_ref, out_ref,
                 xpad_s, u_s, dt_s, b_s, c_s, y_s):
    xv = x_ref[0]
    xz = jax.lax.dot_general(xv, inw_ref[0], (((1,), (1,)), ((), ())),
                             preferred_element_type=jnp.float32) + inb_ref[0]
    xb = xz[:, :D]
    z = xz[:, D:]
    xpad_s[...] = jnp.zeros((HW + 2, HW + 2, D), jnp.float32)
    xpad_s[1:HW + 1, 1:HW + 1, :] = xb.reshape(HW, HW, D)
    cwv = cw_ref[0]
    acc = jnp.zeros((HW, HW, D), jnp.float32)
    for kh in range(3):
        for kw in range(3):
            acc = acc + xpad_s[kh:kh + HW, kw:kw + HW, :] * cwv[kh * 3 + kw][None, None, :]
    xc = _silu(acc + cb_ref[0][None])
    u0 = xc.reshape(L, D)
    rows = jax.lax.broadcasted_iota(jnp.int32, (L, L), 0)
    cols = jax.lax.broadcasted_iota(jnp.int32, (L, L), 1)
    Pm = ((rows % HW) * HW + rows // HW == cols).astype(jnp.float32)
    Fm = (rows + cols == L - 1).astype(jnp.float32)
    uv = jnp.dot(Pm, u0, preferred_element_type=jnp.float32)
    u1 = jnp.dot(Fm, u0, preferred_element_type=jnp.float32)
    u3 = jnp.dot(Fm, uv, preferred_element_type=jnp.float32)
    dirs = [u0, u1, uv, u3]
    for k in range(K):
        uk = dirs[k]
        xdbl = jax.lax.dot_general(uk, xpw_ref[0, k], (((1,), (1,)), ((), ())),
                                   preferred_element_type=jnp.float32)
        dtr = xdbl[:, :R]
        dtl = jax.lax.dot_general(dtr, dtw_ref[0, k], (((1,), (1,)), ((), ())),
                                  preferred_element_type=jnp.float32) + dtb_ref[0, k][None, :]
        dt_s[k, :L, :] = _softplus(dtl)
        b_s[k, :L, :] = xdbl[:, R:R + S]
        c_s[k, :L, :] = xdbl[:, R + S:]
        u_s[k, :L, :] = uk
        dt_s[k, L:, :] = jnp.zeros((LP - L, D), jnp.float32)
        u_s[k, L:, :] = jnp.zeros((LP - L, D), jnp.float32)
        b_s[k, L:, :] = jnp.zeros((LP - L, S), jnp.float32)
        c_s[k, L:, :] = jnp.zeros((LP - L, S), jnp.float32)
    A_all = -jnp.exp(alog_ref[0])

    CH = 16
    def chunk_step(ci, h):
        base = ci * CH
        dtc = dt_s[:, pl.ds(base, CH), :]
        uc = u_s[:, pl.ds(base, CH), :]
        bc = b_s[:, pl.ds(base, CH), :]
        cc = c_s[:, pl.ds(base, CH), :]
        dAc = jnp.exp(dtc[:, :, None, :] * A_all[:, None])
        xin = (dtc * uc)[:, :, None, :] * bc[:, :, :, None]
        ys_ = []
        for j in range(CH):
            h = h * dAc[:, j] + xin[:, j]
            ys_.append(jnp.sum(h * cc[:, j, :, None], axis=1))
        y_s[:, pl.ds(base, CH), :] = jnp.stack(ys_, axis=1)
        return h

    jax.lax.fori_loop(0, LP // CH, chunk_step, jnp.zeros((K, S, D), jnp.float32))
    ysf = y_s[:, :L, :] + dp_ref[0][:, None, :] * u_s[:, :L, :]
    yh = ysf[0] + jnp.dot(Fm, ysf[1], preferred_element_type=jnp.float32)
    yv = ysf[2] + jnp.dot(Fm, ysf[3], preferred_element_type=jnp.float32)
    y = yh + jnp.dot(Pm, yv, preferred_element_type=jnp.float32)
    mu = jnp.mean(y, axis=-1, keepdims=True)
    var = jnp.mean((y - mu) ** 2, axis=-1, keepdims=True)
    yn = (y - mu) / jnp.sqrt(var + 1e-5) * onw_ref[0] + onb_ref[0]
    g = yn * _silu(z)
    pooled = jnp.sum(g, axis=0, keepdims=True) * (1.0 / L)
    mu2 = jnp.mean(pooled, axis=-1, keepdims=True)
    var2 = jnp.mean((pooled - mu2) ** 2, axis=-1, keepdims=True)
    res = (pooled - mu2) / jnp.sqrt(var2 + 1e-5) * lnw_ref[0] + lnb_ref[0]
    out_ref[...] = res.reshape(1, 1, D)


def kernel(x, expert_queries, sa_in_w, sa_in_b, sa_out_w, sa_out_b, ca_in_w,
           ca_in_b, ca_out_w, ca_out_b, in_w, in_b, conv_w, conv_b, xp_w,
           dtp_w, dtp_b, A_log, Dp, on_w, on_b, ln_w, ln_b):
    B = x.shape[0]
    NP = B * TOPK
    cap = float(int(1.25 * B))
    x3 = x.reshape(B, L, D).astype(jnp.float32)
    eq_pad = jnp.zeros((EPAD, D), jnp.float32).at[:E].set(expert_queries)

    full = lambda shp: pl.BlockSpec(shp, lambda i: tuple(0 for _ in shp))
    aw = pl.pallas_call(
        _gate_body,
        grid=(B,),
        in_specs=[
            pl.BlockSpec((1, L, D), lambda i: (i, 0, 0)),
            full((3 * D, D)), full((1, 3 * D)),
            full((D, D)), full((1, D)),
            full((D, D)), full((1, D)),
            full((D, D)), full((1, D)),
            full((EPAD, D)),
        ],
        out_specs=pl.BlockSpec((1, 1, EPAD), lambda i: (i, 0, 0)),
        out_shape=jax.ShapeDtypeStruct((B, 1, EPAD), jnp.float32),
        compiler_params=pltpu.CompilerParams(dimension_semantics=("parallel",)),
    )(x3, sa_in_w, sa_in_b.reshape(1, -1), sa_out_w, sa_out_b.reshape(1, -1),
      ca_in_w[:D], ca_in_b[:D].reshape(1, -1),
      ca_in_w[D:2 * D], ca_in_b[D:2 * D].reshape(1, -1), eq_pad)
    aw = aw.reshape(B, EPAD)

    awT = aw[:, :E].T
    route_fn = functools.partial(
        pl.kernel,
        mesh=plsc.VectorSubcoreMesh(core_axis_name="c", subcore_axis_name="s"),
        out_type=jax.ShapeDtypeStruct((E, B), jnp.float32),
        scratch_types=[
            pltpu.VMEM((E, B), jnp.float32),
            pltpu.VMEM((E, B), jnp.float32),
        ],
    )(functools.partial(_route_sc_body, cap, B))
    route8 = route_fn(awT)
    tks = jnp.stack([route8[0], route8[1]], axis=1)
    tkidx = jnp.stack([route8[2], route8[3]], axis=1).astype(jnp.int32)
    pair_expert = tkidx.reshape(-1)
    aux = route8[4, 0]

    cw = conv_w[:, :, 0].transpose(0, 2, 3, 1).reshape(E, 9, D)
    ew = lambda *dims: pl.BlockSpec((1,) + dims, lambda i, pe: (pe[i],) + tuple(0 for _ in dims))
    outs = pl.pallas_call(
        _expert_body,
        grid_spec=pltpu.PrefetchScalarGridSpec(
            num_scalar_prefetch=1,
            grid=(NP,),
            in_specs=[
                pl.BlockSpec((1, L, D), lambda i, pe: (i // TOPK, 0, 0)),
                ew(2 * D, D), ew(1, 2 * D), ew(9, D), ew(1, D),
                ew(K, R + 2 * S, D), ew(K, D, R), ew(K, D),
                ew(K, S, D), ew(K, D), ew(1, D), ew(1, D), ew(1, D), ew(1, D),
            ],
            out_specs=pl.BlockSpec((1, 1, D), lambda i, pe: (i, 0, 0)),
            scratch_shapes=[
                pltpu.VMEM((HW + 2, HW + 2, D), jnp.float32),
                pltpu.VMEM((K, LP, D), jnp.float32),
                pltpu.VMEM((K, LP, D), jnp.float32),
                pltpu.VMEM((K, LP, S), jnp.float32),
                pltpu.VMEM((K, LP, S), jnp.float32),
                pltpu.VMEM((K, LP, D), jnp.float32),
            ],
        ),
        out_shape=jax.ShapeDtypeStruct((NP, 1, D), jnp.float32),
        compiler_params=pltpu.CompilerParams(dimension_semantics=("parallel",)),
    )(pair_expert, x3, in_w.astype(jnp.float32), in_b.reshape(E, 1, 2 * D),
      cw, conv_b.reshape(E, 1, D), xp_w, dtp_w, dtp_b,
      A_log.transpose(0, 1, 3, 2),
      Dp, on_w.reshape(E, 1, D), on_b.reshape(E, 1, D),
      ln_w.reshape(E, 1, D), ln_b.reshape(E, 1, D))

    mixed = (outs.reshape(NP, D) * tks.reshape(-1, 1)).reshape(B, TOPK, D).sum(axis=1)
    return mixed, aux


# speedup vs baseline: 1.0616x; 1.0616x over previous
import functools

import jax
import jax.numpy as jnp
from jax import lax
from jax.experimental import pallas as pl
from jax.experimental.pallas import tpu as pltpu
from jax.experimental.pallas import tpu_sc as plsc

E, K, S, R, NH, TOPK = 8, 4, 64, 64, 8, 2
HW = 14
L = HW * HW
LP = 200
D = 256
HD = D // NH
EPAD = 128
NEG = -1e30


def _silu(v):
    return v / (1.0 + jnp.exp(-v))


def _softplus(v):
    return jnp.maximum(v, 0.0) + jnp.log(1.0 + jnp.exp(-jnp.abs(v)))


def _gate_body(x_ref, siw_ref, sib_ref, sow_ref, sob_ref,
               cqw_ref, cqb_ref, ckw_ref, ckb_ref, eq_ref, aw_ref):
    xv = x_ref[0]
    qkv = jax.lax.dot_general(xv, siw_ref[...], (((1,), (1,)), ((), ())),
                              preferred_element_type=jnp.float32) + sib_ref[...]
    scale = 1.0 / jnp.sqrt(jnp.float32(HD))
    outs = []
    for h in range(NH):
        q = qkv[:, h * HD:(h + 1) * HD]
        k = qkv[:, D + h * HD:D + (h + 1) * HD]
        v = qkv[:, 2 * D + h * HD:2 * D + (h + 1) * HD]
        s = jax.lax.dot_general(q, k, (((1,), (1,)), ((), ())),
                                preferred_element_type=jnp.float32) * scale
        s = s - jnp.max(s, axis=-1, keepdims=True)
        p = jnp.exp(s)
        p = p / jnp.sum(p, axis=-1, keepdims=True)
        outs.append(jnp.dot(p, v, preferred_element_type=jnp.float32))
    ctx0 = jnp.concatenate(outs, axis=1)
    ctx = jax.lax.dot_general(ctx0, sow_ref[...], (((1,), (1,)), ((), ())),
                              preferred_element_type=jnp.float32) + sob_ref[...]
    q2 = jax.lax.dot_general(ctx, cqw_ref[...], (((1,), (1,)), ((), ())),
                             preferred_element_type=jnp.float32) + cqb_ref[...]
    k2 = jax.lax.dot_general(eq_ref[...], ckw_ref[...], (((1,), (1,)), ((), ())),
                             preferred_element_type=jnp.float32) + ckb_ref[...]
    col = jax.lax.broadcasted_iota(jnp.int32, (L, EPAD), 1)
    acc = jnp.zeros((L, EPAD), jnp.float32)
    for h in range(NH):
        q2h = q2[:, h * HD:(h + 1) * HD]
        k2h = k2[:, h * HD:(h + 1) * HD]
        l2 = jax.lax.dot_general(q2h, k2h, (((1,), (1,)), ((), ())),
                                 preferred_element_type=jnp.float32) * scale
        l2 = jnp.where(col < E, l2, NEG)
        l2 = l2 - jnp.max(l2, axis=-1, keepdims=True)
        p2 = jnp.exp(l2)
        p2 = p2 / jnp.sum(p2, axis=-1, keepdims=True)
        acc = acc + p2
    aw_ref[...] = (jnp.sum(acc, axis=0) * (1.0 / (NH * L))).reshape(1, 1, EPAD)


def _route_sc_body(cap, nb, awT_hbm, out_hbm, aw_v, out_v):
    c0 = lax.axis_index("c") == 0
    s0 = lax.axis_index("s") == 0

    def vsum(v):
        idx = lax.iota(jnp.int32, nb)
        for k in (8, 4, 2, 1):
            v = v + v.at[jnp.bitwise_xor(idx, k)].get(
                mode="promise_in_bounds", unique_indices=True)
        return v

    @pl.when(jnp.logical_and(c0, s0))
    def _():
        pltpu.sync_copy(awT_hbm, aw_v)
        cols = [aw_v[e] for e in range(E)]
        m = cols[0]
        for e in range(1, E):
            m = jnp.maximum(m, cols[e])
        exs = [jnp.exp(c - m) for c in cols]
        ssum = exs[0]
        for e in range(1, E):
            ssum = ssum + exs[e]
        raw = [ex / ssum for ex in exs]

        def top2(vals):
            b1 = vals[0]
            i1 = jnp.zeros((nb,), jnp.float32)
            for e in range(1, E):
                upd = vals[e] > b1
                b1 = jnp.where(upd, vals[e], b1)
                i1 = jnp.where(upd, float(e), i1)
            b2 = jnp.full((nb,), -1.0, jnp.float32)
            i2 = jnp.zeros((nb,), jnp.float32)
            for e in range(E):
                upd = jnp.logical_and(i1 != float(e), vals[e] > b2)
                b2 = jnp.where(upd, vals[e], b2)
                i2 = jnp.where(upd, float(e), i2)
            return b1, i1, b2, i2

        _, i1, _, i2 = top2(raw)
        masks = [jnp.where(jnp.logical_or(i1 == float(e), i2 == float(e)),
                           1.0, 0.0) for e in range(E)]
        gates = []
        auxs = jnp.zeros((nb,), jnp.float32)
        for e in range(E):
            me = raw[e] * masks[e]
            gates.append(me / (vsum(me) + 1e-6) * cap)
            auxs = auxs + vsum(raw[e]) * vsum(masks[e])
        g1, j1, g2, j2 = top2(gates)
        e2v = jnp.exp(g2 - g1)
        t1 = 1.0 / (1.0 + e2v)
        t2 = e2v / (1.0 + e2v)
        out_v[0] = t1
        out_v[1] = t2
        out_v[2] = j1
        out_v[3] = j2
        out_v[4] = auxs * (0.01 * E / (nb * nb))
        zero = jnp.zeros((nb,), jnp.float32)
        out_v[5] = zero
        out_v[6] = zero
        out_v[7] = zero
        pltpu.sync_copy(out_v, out_hbm)


def _route_body(cap, nb, aw_ref, out_ref):
    aw = aw_ref[...]
    col = jax.lax.broadcasted_iota(jnp.int32, (nb, EPAD), 1)
    valid = col < E
    logits = jnp.where(valid, aw, NEG)
    m = jnp.max(logits, -1, keepdims=True)
    ex = jnp.exp(logits - m)
    raw = ex / jnp.sum(ex, -1, keepdims=True)

    def top1(vals):
        mv = jnp.max(vals, -1, keepdims=True)
        idx = jnp.min(jnp.where(vals == mv, col, 1 << 20), -1, keepdims=True)
        return mv, idx

    sel = jnp.where(valid, raw, -1.0)
    _, i1 = top1(sel)
    sel2 = jnp.where(col == i1, -1.0, sel)
    _, i2 = top1(sel2)
    maskf = ((col == i1) | (col == i2)).astype(jnp.float32)
    masked = raw * maskf
    colsum = jnp.sum(masked, axis=0, keepdims=True)
    gate = masked / (colsum + 1e-6) * cap
    gsel = jnp.where(valid, gate, -1.0)
    g1, j1 = top1(gsel)
    gsel2 = jnp.where(col == j1, -1.0, gsel)
    g2, j2 = top1(gsel2)
    e2 = jnp.exp(g2 - g1)
    t1 = 1.0 / (1.0 + e2)
    t2 = e2 / (1.0 + e2)
    rm = jnp.sum(raw, axis=0, keepdims=True) * (1.0 / nb)
    mm = jnp.sum(maskf, axis=0, keepdims=True) * (1.0 / nb)
    aux = 0.01 * jnp.sum(rm * mm) * E
    out = (jnp.where(col == 0, t1, 0.0)
           + jnp.where(col == 1, t2, 0.0)
           + jnp.where(col == 2, j1.astype(jnp.float32), 0.0)
           + jnp.where(col == 3, j2.astype(jnp.float32), 0.0)
           + jnp.where(col == 4, aux, 0.0))
    out_ref[...] = out


def _expert_body(pe_ref, x_ref, inw_ref, inb_ref, cw_ref, cb_ref, xpw_ref,
                 dtw_ref, dtb_ref, alog_ref, dp_ref, onw_ref, onb_ref,
                 lnw_ref, lnb_ref, out_ref,
                 xpad_s, u_s, dt_s, b_s, c_s, y_s):
    xv = x_ref[0]
    xz = jax.lax.dot_general(xv, inw_ref[0], (((1,), (1,)), ((), ())),
                             preferred_element_type=jnp.float32) + inb_ref[0]
    xb = xz[:, :D]
    z = xz[:, D:]
    xpad_s[...] = jnp.zeros((HW + 2, HW + 2, D), jnp.float32)
    xpad_s[1:HW + 1, 1:HW + 1, :] = xb.reshape(HW, HW, D)
    cwv = cw_ref[0]
    acc = jnp.zeros((HW, HW, D), jnp.float32)
    for kh in range(3):
        for kw in range(3):
            acc = acc + xpad_s[kh:kh + HW, kw:kw + HW, :] * cwv[kh * 3 + kw][None, None, :]
    xc = _silu(acc + cb_ref[0][None])
    u0 = xc.reshape(L, D)
    rows = jax.lax.broadcasted_iota(jnp.int32, (L, L), 0)
    cols = jax.lax.broadcasted_iota(jnp.int32, (L, L), 1)
    Pm = ((rows % HW) * HW + rows // HW == cols).astype(jnp.float32)
    Fm = (rows + cols == L - 1).astype(jnp.float32)
    uv = jnp.dot(Pm, u0, preferred_element_type=jnp.float32)
    u1 = jnp.dot(Fm, u0, preferred_element_type=jnp.float32)
    u3 = jnp.dot(Fm, uv, preferred_element_type=jnp.float32)
    dirs = [u0, u1, uv, u3]
    for k in range(K):
        uk = dirs[k]
        xdbl = jax.lax.dot_general(uk, xpw_ref[0, k], (((1,), (1,)), ((), ())),
                                   preferred_element_type=jnp.float32)
        dtr = xdbl[:, :R]
        dtl = jax.lax.dot_general(dtr, dtw_ref[0, k], (((1,), (1,)), ((), ())),
                                  preferred_element_type=jnp.float32) + dtb_ref[0, k][None, :]
        dt_s[k, :L, :] = _softplus(dtl)
        b_s[k, :L, :] = xdbl[:, R:R + S]
        c_s[k, :L, :] = xdbl[:, R + S:]
        u_s[k, :L, :] = uk
        dt_s[k, L:, :] = jnp.zeros((LP - L, D), jnp.float32)
        u_s[k, L:, :] = jnp.zeros((LP - L, D), jnp.float32)
        b_s[k, L:, :] = jnp.zeros((LP - L, S), jnp.float32)
        c_s[k, L:, :] = jnp.zeros((LP - L, S), jnp.float32)
    A2 = -jnp.exp(alog_ref[0]) * jnp.float32(1.4426950408889634)

    CH = 8
    def chunk_step(ci, h):
        base = ci * CH
        dtc = dt_s[:, pl.ds(base, CH), :]
        uc = u_s[:, pl.ds(base, CH), :]
        bc = b_s[:, pl.ds(base, CH), :]
        cc = c_s[:, pl.ds(base, CH), :]
        dAc = jnp.exp2(dtc[:, :, None, :] * A2[:, None])
        xin = (dtc * uc)[:, :, None, :] * bc[:, :, :, None]
        ys_ = []
        for j in range(CH):
            h = h * dAc[:, j] + xin[:, j]
            ys_.append(jnp.sum(h * cc[:, j, :, None], axis=1))
        y_s[:, pl.ds(base, CH), :] = jnp.stack(ys_, axis=1)
        return h

    jax.lax.fori_loop(0, LP // CH, chunk_step, jnp.zeros((K, S, D), jnp.float32))
    ysf = y_s[:, :L, :] + dp_ref[0][:, None, :] * u_s[:, :L, :]
    yh = ysf[0] + jnp.dot(Fm, ysf[1], preferred_element_type=jnp.float32)
    yv = ysf[2] + jnp.dot(Fm, ysf[3], preferred_element_type=jnp.float32)
    y = yh + jnp.dot(Pm, yv, preferred_element_type=jnp.float32)
    mu = jnp.mean(y, axis=-1, keepdims=True)
    var = jnp.mean((y - mu) ** 2, axis=-1, keepdims=True)
    yn = (y - mu) / jnp.sqrt(var + 1e-5) * onw_ref[0] + onb_ref[0]
    g = yn * _silu(z)
    pooled = jnp.sum(g, axis=0, keepdims=True) * (1.0 / L)
    mu2 = jnp.mean(pooled, axis=-1, keepdims=True)
    var2 = jnp.mean((pooled - mu2) ** 2, axis=-1, keepdims=True)
    res = (pooled - mu2) / jnp.sqrt(var2 + 1e-5) * lnw_ref[0] + lnb_ref[0]
    out_ref[...] = res.reshape(1, 1, D)


def kernel(x, expert_queries, sa_in_w, sa_in_b, sa_out_w, sa_out_b, ca_in_w,
           ca_in_b, ca_out_w, ca_out_b, in_w, in_b, conv_w, conv_b, xp_w,
           dtp_w, dtp_b, A_log, Dp, on_w, on_b, ln_w, ln_b):
    B = x.shape[0]
    NP = B * TOPK
    cap = float(int(1.25 * B))
    x3 = x.reshape(B, L, D).astype(jnp.float32)
    eq_pad = jnp.zeros((EPAD, D), jnp.float32).at[:E].set(expert_queries)

    full = lambda shp: pl.BlockSpec(shp, lambda i: tuple(0 for _ in shp))
    aw = pl.pallas_call(
        _gate_body,
        grid=(B,),
        in_specs=[
            pl.BlockSpec((1, L, D), lambda i: (i, 0, 0)),
            full((3 * D, D)), full((1, 3 * D)),
            full((D, D)), full((1, D)),
            full((D, D)), full((1, D)),
            full((D, D)), full((1, D)),
            full((EPAD, D)),
        ],
        out_specs=pl.BlockSpec((1, 1, EPAD), lambda i: (i, 0, 0)),
        out_shape=jax.ShapeDtypeStruct((B, 1, EPAD), jnp.float32),
        compiler_params=pltpu.CompilerParams(dimension_semantics=("parallel",)),
    )(x3, sa_in_w, sa_in_b.reshape(1, -1), sa_out_w, sa_out_b.reshape(1, -1),
      ca_in_w[:D], ca_in_b[:D].reshape(1, -1),
      ca_in_w[D:2 * D], ca_in_b[D:2 * D].reshape(1, -1), eq_pad)
    aw = aw.reshape(B, EPAD)

    awT = aw[:, :E].T
    route_fn = functools.partial(
        pl.kernel,
        mesh=plsc.VectorSubcoreMesh(core_axis_name="c", subcore_axis_name="s"),
        out_type=jax.ShapeDtypeStruct((E, B), jnp.float32),
        scratch_types=[
            pltpu.VMEM((E, B), jnp.float32),
            pltpu.VMEM((E, B), jnp.float32),
        ],
    )(functools.partial(_route_sc_body, cap, B))
    route8 = route_fn(awT)
    tks = jnp.stack([route8[0], route8[1]], axis=1)
    tkidx = jnp.stack([route8[2], route8[3]], axis=1).astype(jnp.int32)
    pair_expert = tkidx.reshape(-1)
    aux = route8[4, 0]

    cw = conv_w[:, :, 0].transpose(0, 2, 3, 1).reshape(E, 9, D)
    ew = lambda *dims: pl.BlockSpec((1,) + dims, lambda i, pe: (pe[i],) + tuple(0 for _ in dims))
    outs = pl.pallas_call(
        _expert_body,
        grid_spec=pltpu.PrefetchScalarGridSpec(
            num_scalar_prefetch=1,
            grid=(NP,),
            in_specs=[
                pl.BlockSpec((1, L, D), lambda i, pe: (i // TOPK, 0, 0)),
                ew(2 * D, D), ew(1, 2 * D), ew(9, D), ew(1, D),
                ew(K, R + 2 * S, D), ew(K, D, R), ew(K, D),
                ew(K, S, D), ew(K, D), ew(1, D), ew(1, D), ew(1, D), ew(1, D),
            ],
            out_specs=pl.BlockSpec((1, 1, D), lambda i, pe: (i, 0, 0)),
            scratch_shapes=[
                pltpu.VMEM((HW + 2, HW + 2, D), jnp.float32),
                pltpu.VMEM((K, LP, D), jnp.float32),
                pltpu.VMEM((K, LP, D), jnp.float32),
                pltpu.VMEM((K, LP, S), jnp.float32),
                pltpu.VMEM((K, LP, S), jnp.float32),
                pltpu.VMEM((K, LP, D), jnp.float32),
            ],
        ),
        out_shape=jax.ShapeDtypeStruct((NP, 1, D), jnp.float32),
        compiler_params=pltpu.CompilerParams(dimension_semantics=("parallel",)),
    )(pair_expert, x3, in_w.astype(jnp.float32), in_b.reshape(E, 1, 2 * D),
      cw, conv_b.reshape(E, 1, D), xp_w, dtp_w, dtp_b,
      A_log.transpose(0, 1, 3, 2),
      Dp, on_w.reshape(E, 1, D), on_b.reshape(E, 1, D),
      ln_w.reshape(E, 1, D), ln_b.reshape(E, 1, D))

    mixed = (outs.reshape(NP, D) * tks.reshape(-1, 1)).reshape(B, TOPK, D).sum(axis=1)
    return mixed, aux


# speedup vs baseline: 1.1328x; 1.0671x over previous
import functools

import jax
import jax.numpy as jnp
from jax import lax
from jax.experimental import pallas as pl
from jax.experimental.pallas import tpu as pltpu
from jax.experimental.pallas import tpu_sc as plsc

E, K, S, R, NH, TOPK = 8, 4, 64, 64, 8, 2
HW = 14
L = HW * HW
LP = 200
D = 256
HD = D // NH
EPAD = 128
NEG = -1e30


def _silu(v):
    return v / (1.0 + jnp.exp(-v))


def _softplus(v):
    return jnp.maximum(v, 0.0) + jnp.log(1.0 + jnp.exp(-jnp.abs(v)))


def _gate_body(x_ref, siw_ref, sib_ref, sow_ref, sob_ref,
               cqw_ref, cqb_ref, ckw_ref, ckb_ref, eq_ref, aw_ref):
    xv = x_ref[0]
    qkv = jax.lax.dot_general(xv, siw_ref[...], (((1,), (1,)), ((), ())),
                              preferred_element_type=jnp.float32) + sib_ref[...]
    scale = 1.0 / jnp.sqrt(jnp.float32(HD))
    outs = []
    for h in range(NH):
        q = qkv[:, h * HD:(h + 1) * HD]
        k = qkv[:, D + h * HD:D + (h + 1) * HD]
        v = qkv[:, 2 * D + h * HD:2 * D + (h + 1) * HD]
        s = jax.lax.dot_general(q, k, (((1,), (1,)), ((), ())),
                                preferred_element_type=jnp.float32) * scale
        s = s - jnp.max(s, axis=-1, keepdims=True)
        p = jnp.exp(s)
        p = p / jnp.sum(p, axis=-1, keepdims=True)
        outs.append(jnp.dot(p, v, preferred_element_type=jnp.float32))
    ctx0 = jnp.concatenate(outs, axis=1)
    ctx = jax.lax.dot_general(ctx0, sow_ref[...], (((1,), (1,)), ((), ())),
                              preferred_element_type=jnp.float32) + sob_ref[...]
    q2 = jax.lax.dot_general(ctx, cqw_ref[...], (((1,), (1,)), ((), ())),
                             preferred_element_type=jnp.float32) + cqb_ref[...]
    k2 = jax.lax.dot_general(eq_ref[...], ckw_ref[...], (((1,), (1,)), ((), ())),
                             preferred_element_type=jnp.float32) + ckb_ref[...]
    col = jax.lax.broadcasted_iota(jnp.int32, (L, EPAD), 1)
    acc = jnp.zeros((L, EPAD), jnp.float32)
    for h in range(NH):
        q2h = q2[:, h * HD:(h + 1) * HD]
        k2h = k2[:, h * HD:(h + 1) * HD]
        l2 = jax.lax.dot_general(q2h, k2h, (((1,), (1,)), ((), ())),
                                 preferred_element_type=jnp.float32) * scale
        l2 = jnp.where(col < E, l2, NEG)
        l2 = l2 - jnp.max(l2, axis=-1, keepdims=True)
        p2 = jnp.exp(l2)
        p2 = p2 / jnp.sum(p2, axis=-1, keepdims=True)
        acc = acc + p2
    aw_ref[...] = (jnp.sum(acc, axis=0) * (1.0 / (NH * L))).reshape(1, 1, EPAD)


def _route_sc_body(cap, nb, awT_hbm, out_hbm, aw_v, out_v):
    c0 = lax.axis_index("c") == 0
    s0 = lax.axis_index("s") == 0

    def vsum(v):
        idx = lax.iota(jnp.int32, nb)
        for k in (8, 4, 2, 1):
            v = v + v.at[jnp.bitwise_xor(idx, k)].get(
                mode="promise_in_bounds", unique_indices=True)
        return v

    @pl.when(jnp.logical_and(c0, s0))
    def _():
        pltpu.sync_copy(awT_hbm, aw_v)
        cols = [aw_v[e] for e in range(E)]
        m = cols[0]
        for e in range(1, E):
            m = jnp.maximum(m, cols[e])
        exs = [jnp.exp(c - m) for c in cols]
        ssum = exs[0]
        for e in range(1, E):
            ssum = ssum + exs[e]
        raw = [ex / ssum for ex in exs]

        def top2(vals):
            b1 = vals[0]
            i1 = jnp.zeros((nb,), jnp.float32)
            for e in range(1, E):
                upd = vals[e] > b1
                b1 = jnp.where(upd, vals[e], b1)
                i1 = jnp.where(upd, float(e), i1)
            b2 = jnp.full((nb,), -1.0, jnp.float32)
            i2 = jnp.zeros((nb,), jnp.float32)
            for e in range(E):
                upd = jnp.logical_and(i1 != float(e), vals[e] > b2)
                b2 = jnp.where(upd, vals[e], b2)
                i2 = jnp.where(upd, float(e), i2)
            return b1, i1, b2, i2

        _, i1, _, i2 = top2(raw)
        masks = [jnp.where(jnp.logical_or(i1 == float(e), i2 == float(e)),
                           1.0, 0.0) for e in range(E)]
        gates = []
        auxs = jnp.zeros((nb,), jnp.float32)
        for e in range(E):
            me = raw[e] * masks[e]
            gates.append(me / (vsum(me) + 1e-6) * cap)
            auxs = auxs + vsum(raw[e]) * vsum(masks[e])
        g1, j1, g2, j2 = top2(gates)
        e2v = jnp.exp(g2 - g1)
        t1 = 1.0 / (1.0 + e2v)
        t2 = e2v / (1.0 + e2v)
        out_v[0] = t1
        out_v[1] = t2
        out_v[2] = j1
        out_v[3] = j2
        out_v[4] = auxs * (0.01 * E / (nb * nb))
        zero = jnp.zeros((nb,), jnp.float32)
        out_v[5] = zero
        out_v[6] = zero
        out_v[7] = zero
        pltpu.sync_copy(out_v, out_hbm)


def _route_body(cap, nb, aw_ref, out_ref):
    aw = aw_ref[...]
    col = jax.lax.broadcasted_iota(jnp.int32, (nb, EPAD), 1)
    valid = col < E
    logits = jnp.where(valid, aw, NEG)
    m = jnp.max(logits, -1, keepdims=True)
    ex = jnp.exp(logits - m)
    raw = ex / jnp.sum(ex, -1, keepdims=True)

    def top1(vals):
        mv = jnp.max(vals, -1, keepdims=True)
        idx = jnp.min(jnp.where(vals == mv, col, 1 << 20), -1, keepdims=True)
        return mv, idx

    sel = jnp.where(valid, raw, -1.0)
    _, i1 = top1(sel)
    sel2 = jnp.where(col == i1, -1.0, sel)
    _, i2 = top1(sel2)
    maskf = ((col == i1) | (col == i2)).astype(jnp.float32)
    masked = raw * maskf
    colsum = jnp.sum(masked, axis=0, keepdims=True)
    gate = masked / (colsum + 1e-6) * cap
    gsel = jnp.where(valid, gate, -1.0)
    g1, j1 = top1(gsel)
    gsel2 = jnp.where(col == j1, -1.0, gsel)
    g2, j2 = top1(gsel2)
    e2 = jnp.exp(g2 - g1)
    t1 = 1.0 / (1.0 + e2)
    t2 = e2 / (1.0 + e2)
    rm = jnp.sum(raw, axis=0, keepdims=True) * (1.0 / nb)
    mm = jnp.sum(maskf, axis=0, keepdims=True) * (1.0 / nb)
    aux = 0.01 * jnp.sum(rm * mm) * E
    out = (jnp.where(col == 0, t1, 0.0)
           + jnp.where(col == 1, t2, 0.0)
           + jnp.where(col == 2, j1.astype(jnp.float32), 0.0)
           + jnp.where(col == 3, j2.astype(jnp.float32), 0.0)
           + jnp.where(col == 4, aux, 0.0))
    out_ref[...] = out


def _expert_body(pe_ref, x_ref, inw_ref, inb_ref, cw_ref, cb_ref, xpw_ref,
                 dtw_ref, dtb_ref, alog_ref, dp_ref, onw_ref, onb_ref,
                 lnw_ref, lnb_ref, out_ref,
                 xpad_s, u_s, dt_s, b_s, c_s, y_s):
    xv = x_ref[0]
    xz = jax.lax.dot_general(xv, inw_ref[0], (((1,), (1,)), ((), ())),
                             preferred_element_type=jnp.float32) + inb_ref[0]
    xb = xz[:, :D]
    z = xz[:, D:]
    xpad_s[...] = jnp.zeros((HW + 2, HW + 2, D), jnp.float32)
    xpad_s[1:HW + 1, 1:HW + 1, :] = xb.reshape(HW, HW, D)
    cwv = cw_ref[0]
    acc = jnp.zeros((HW, HW, D), jnp.float32)
    for kh in range(3):
        for kw in range(3):
            acc = acc + xpad_s[kh:kh + HW, kw:kw + HW, :] * cwv[kh * 3 + kw][None, None, :]
    xc = _silu(acc + cb_ref[0][None])
    u0 = xc.reshape(L, D)
    rows = jax.lax.broadcasted_iota(jnp.int32, (L, L), 0)
    cols = jax.lax.broadcasted_iota(jnp.int32, (L, L), 1)
    Pm = ((rows % HW) * HW + rows // HW == cols).astype(jnp.float32)
    Fm = (rows + cols == L - 1).astype(jnp.float32)
    uv = jnp.dot(Pm, u0, preferred_element_type=jnp.float32)
    u1 = jnp.dot(Fm, u0, preferred_element_type=jnp.float32)
    u3 = jnp.dot(Fm, uv, preferred_element_type=jnp.float32)
    dirs = [u0, u1, uv, u3]
    for k in range(K):
        uk = dirs[k]
        xdbl = jax.lax.dot_general(uk, xpw_ref[0, k], (((1,), (1,)), ((), ())),
                                   preferred_element_type=jnp.float32)
        dtr = xdbl[:, :R]
        dtl = jax.lax.dot_general(dtr, dtw_ref[0, k], (((1,), (1,)), ((), ())),
                                  preferred_element_type=jnp.float32) + dtb_ref[0, k][None, :]
        dt_s[k, :L, :] = _softplus(dtl)
        b_s[k, :L, :] = xdbl[:, R:R + S]
        c_s[k, :L, :] = xdbl[:, R + S:]
        u_s[k, :L, :] = uk
        dt_s[k, L:, :] = jnp.zeros((LP - L, D), jnp.float32)
        u_s[k, L:, :] = jnp.zeros((LP - L, D), jnp.float32)
        b_s[k, L:, :] = jnp.zeros((LP - L, S), jnp.float32)
        c_s[k, L:, :] = jnp.zeros((LP - L, S), jnp.float32)
    A2 = -jnp.exp(alog_ref[0]) * jnp.float32(1.4426950408889634)

    CH = 8
    def chunk_step(base, h):
        dtc = dt_s[:, pl.ds(base, CH), :]
        uc = u_s[:, pl.ds(base, CH), :]
        bc = b_s[:, pl.ds(base, CH), :]
        cc = c_s[:, pl.ds(base, CH), :]
        dAc = jnp.exp2(dtc[:, :, None, :] * A2[:, None])
        xin = (dtc * uc)[:, :, None, :] * bc[:, :, :, None]
        ys_ = []
        for j in range(CH):
            h = h * dAc[:, j] + xin[:, j]
            ys_.append(jnp.sum(h * cc[:, j, :, None], axis=1))
        y_s[:, pl.ds(base, CH), :] = jnp.stack(ys_, axis=1)
        return h

    h = jnp.zeros((K, S, D), jnp.float32)
    for ci in range(L // CH + 1):
        h = chunk_step(ci * CH, h)
    ysf = y_s[:, :L, :] + dp_ref[0][:, None, :] * u_s[:, :L, :]
    yh = ysf[0] + jnp.dot(Fm, ysf[1], preferred_element_type=jnp.float32)
    yv = ysf[2] + jnp.dot(Fm, ysf[3], preferred_element_type=jnp.float32)
    y = yh + jnp.dot(Pm, yv, preferred_element_type=jnp.float32)
    mu = jnp.mean(y, axis=-1, keepdims=True)
    var = jnp.mean((y - mu) ** 2, axis=-1, keepdims=True)
    yn = (y - mu) / jnp.sqrt(var + 1e-5) * onw_ref[0] + onb_ref[0]
    g = yn * _silu(z)
    pooled = jnp.sum(g, axis=0, keepdims=True) * (1.0 / L)
    mu2 = jnp.mean(pooled, axis=-1, keepdims=True)
    var2 = jnp.mean((pooled - mu2) ** 2, axis=-1, keepdims=True)
    res = (pooled - mu2) / jnp.sqrt(var2 + 1e-5) * lnw_ref[0] + lnb_ref[0]
    out_ref[...] = res.reshape(1, 1, D)


def kernel(x, expert_queries, sa_in_w, sa_in_b, sa_out_w, sa_out_b, ca_in_w,
           ca_in_b, ca_out_w, ca_out_b, in_w, in_b, conv_w, conv_b, xp_w,
           dtp_w, dtp_b, A_log, Dp, on_w, on_b, ln_w, ln_b):
    B = x.shape[0]
    NP = B * TOPK
    cap = float(int(1.25 * B))
    x3 = x.reshape(B, L, D).astype(jnp.float32)
    eq_pad = jnp.zeros((EPAD, D), jnp.float32).at[:E].set(expert_queries)

    full = lambda shp: pl.BlockSpec(shp, lambda i: tuple(0 for _ in shp))
    aw = pl.pallas_call(
        _gate_body,
        grid=(B,),
        in_specs=[
            pl.BlockSpec((1, L, D), lambda i: (i, 0, 0)),
            full((3 * D, D)), full((1, 3 * D)),
            full((D, D)), full((1, D)),
            full((D, D)), full((1, D)),
            full((D, D)), full((1, D)),
            full((EPAD, D)),
        ],
        out_specs=pl.BlockSpec((1, 1, EPAD), lambda i: (i, 0, 0)),
        out_shape=jax.ShapeDtypeStruct((B, 1, EPAD), jnp.float32),
        compiler_params=pltpu.CompilerParams(dimension_semantics=("parallel",)),
    )(x3, sa_in_w, sa_in_b.reshape(1, -1), sa_out_w, sa_out_b.reshape(1, -1),
      ca_in_w[:D], ca_in_b[:D].reshape(1, -1),
      ca_in_w[D:2 * D], ca_in_b[D:2 * D].reshape(1, -1), eq_pad)
    aw = aw.reshape(B, EPAD)

    awT = aw[:, :E].T
    route_fn = functools.partial(
        pl.kernel,
        mesh=plsc.VectorSubcoreMesh(core_axis_name="c", subcore_axis_name="s"),
        out_type=jax.ShapeDtypeStruct((E, B), jnp.float32),
        scratch_types=[
            pltpu.VMEM((E, B), jnp.float32),
            pltpu.VMEM((E, B), jnp.float32),
        ],
    )(functools.partial(_route_sc_body, cap, B))
    route8 = route_fn(awT)
    tks = jnp.stack([route8[0], route8[1]], axis=1)
    tkidx = jnp.stack([route8[2], route8[3]], axis=1).astype(jnp.int32)
    pair_expert = tkidx.reshape(-1)
    aux = route8[4, 0]

    cw = conv_w[:, :, 0].transpose(0, 2, 3, 1).reshape(E, 9, D)
    ew = lambda *dims: pl.BlockSpec((1,) + dims, lambda i, pe: (pe[i],) + tuple(0 for _ in dims))
    outs = pl.pallas_call(
        _expert_body,
        grid_spec=pltpu.PrefetchScalarGridSpec(
            num_scalar_prefetch=1,
            grid=(NP,),
            in_specs=[
                pl.BlockSpec((1, L, D), lambda i, pe: (i // TOPK, 0, 0)),
                ew(2 * D, D), ew(1, 2 * D), ew(9, D), ew(1, D),
                ew(K, R + 2 * S, D), ew(K, D, R), ew(K, D),
                ew(K, S, D), ew(K, D), ew(1, D), ew(1, D), ew(1, D), ew(1, D),
            ],
            out_specs=pl.BlockSpec((1, 1, D), lambda i, pe: (i, 0, 0)),
            scratch_shapes=[
                pltpu.VMEM((HW + 2, HW + 2, D), jnp.float32),
                pltpu.VMEM((K, LP, D), jnp.float32),
                pltpu.VMEM((K, LP, D), jnp.float32),
                pltpu.VMEM((K, LP, S), jnp.float32),
                pltpu.VMEM((K, LP, S), jnp.float32),
                pltpu.VMEM((K, LP, D), jnp.float32),
            ],
        ),
        out_shape=jax.ShapeDtypeStruct((NP, 1, D), jnp.float32),
        compiler_params=pltpu.CompilerParams(dimension_semantics=("parallel",)),
    )(pair_expert, x3, in_w.astype(jnp.float32), in_b.reshape(E, 1, 2 * D),
      cw, conv_b.reshape(E, 1, D), xp_w, dtp_w, dtp_b,
      A_log.transpose(0, 1, 3, 2),
      Dp, on_w.reshape(E, 1, D), on_b.reshape(E, 1, D),
      ln_w.reshape(E, 1, D), ln_b.reshape(E, 1, D))

    mixed = (outs.reshape(NP, D) * tks.reshape(-1, 1)).reshape(B, TOPK, D).sum(axis=1)
    return mixed, aux
